# baseline (speedup 1.0000x reference)
import functools

import jax
import jax.numpy as jnp
from jax import lax
from jax.experimental import pallas as pl
from jax.experimental.pallas import tpu as pltpu
from jax.experimental.pallas import tpu_sc as plsc

N = 1024
D = 128
MY = 100000
K = 30
BLK = 128
NB = 784
MP = NB * BLK
CB = 1024
GSTEPS = MP // CB
KP = 32
CAND = K * BLK
EPS = 1e-8

NC = 2
NS = 16
NW = NC * NS
BPW = (N * K) // NW
CH = 8
CW = BPW // CH


def _scores_body(x_ref, y_ref, s_ref, m_ref, xn_ref):
    g = pl.program_id(0)

    @pl.when(g == 0)
    def _():
        x = x_ref[...]
        nrm = jnp.sqrt(jnp.sum(x * x, axis=1, keepdims=True))
        xn_ref[...] = x / jnp.maximum(nrm, EPS)

    y = y_ref[...]
    ynrm = jnp.sqrt(jnp.sum(y * y, axis=1, keepdims=True))
    yn = y / jnp.maximum(ynrm, EPS)
    s = lax.dot_general(xn_ref[...], yn, (((1,), (1,)), ((), ())),
                        preferred_element_type=jnp.float32)

    def _store(sv):
        for b in range(CB // BLK):
            sb = sv[:, b * BLK:(b + 1) * BLK]
            s_ref[b] = sb
            m_ref[0, :, b:b + 1] = jnp.max(sb, axis=1, keepdims=True)

    @pl.when(g < GSTEPS - 1)
    def _():
        _store(s)

    @pl.when(g == GSTEPS - 1)
    def _():
        cols = g * CB + lax.broadcasted_iota(jnp.int32, (N, CB), 1)
        _store(jnp.where(cols < MY, s, -jnp.inf))


def _compute_scores(x, y):
    return pl.pallas_call(
        _scores_body,
        grid=(GSTEPS,),
        in_specs=[
            pl.BlockSpec((N, D), lambda g: (0, 0)),
            pl.BlockSpec((CB, D), lambda g: (g, 0)),
        ],
        out_specs=[
            pl.BlockSpec((CB // BLK, N, BLK), lambda g: (g, 0, 0)),
            pl.BlockSpec((1, N, CB // BLK), lambda g: (g, 0, 0)),
        ],
        out_shape=[
            jax.ShapeDtypeStruct((NB, N, BLK), jnp.float32),
            jax.ShapeDtypeStruct((GSTEPS, N, CB // BLK), jnp.float32),
        ],
        scratch_shapes=[pltpu.VMEM((N, D), jnp.float32)],
    )(x, y)


_MPAD = 896


def _blocksel_body(m_ref, g_ref):
    m = m_ref[...]
    iota = lax.broadcasted_iota(jnp.int32, (N, _MPAD), 1)
    rowc = lax.broadcasted_iota(jnp.int32, (N, 1), 0)
    g_ref[...] = lax.broadcasted_iota(jnp.int32, (N, KP), 0)
    big = jnp.int32(1 << 30)
    for k in range(K):
        mx = jnp.max(m, axis=1, keepdims=True)
        idx = jnp.min(jnp.where(m == mx, iota, big), axis=1, keepdims=True)
        g_ref[:, k:k + 1] = idx * N + rowc
        m = jnp.where(iota == idx, -jnp.inf, m)


def _select_blocks(mpad):
    return pl.pallas_call(
        _blocksel_body,
        out_shape=jax.ShapeDtypeStruct((N, KP), jnp.int32),
    )(mpad)


def _gather_body(s2_ref, idx_ref, out_ref, idx_v, rows_v, sem):
    wid = lax.axis_index("s") * NC + lax.axis_index("c")
    pltpu.sync_copy(idx_ref.at[wid], idx_v)
    cps = [pltpu.async_copy(s2_ref.at[idx_v.at[j]], rows_v.at[j], sem)
           for j in range(CH)]
    for cp in cps:
        cp.wait()
    pltpu.sync_copy(rows_v, out_ref.at[wid])


def _sc_gather(s2, idx3):
    mesh = plsc.VectorSubcoreMesh(core_axis_name="c", subcore_axis_name="s")
    fn = functools.partial(
        pl.kernel,
        mesh=mesh,
        out_type=jax.ShapeDtypeStruct((NW, CH, CW, BLK), jnp.float32),
        scratch_types=[
            pltpu.VMEM((CH, CW), jnp.int32),
            pltpu.VMEM((CH, CW, BLK), jnp.float32),
            pltpu.SemaphoreType.DMA,
        ],
    )(_gather_body)
    return fn(s2, idx3)


_RT = 128


def _final_body(c_ref, g_ref, vals_ref, idx_ref):
    c = c_ref[...]
    gsel = g_ref[...]
    blk_f = (gsel // N).astype(jnp.float32)
    je = lax.broadcasted_iota(jnp.int32, (KP, CAND), 1) // BLK
    ie = lax.broadcasted_iota(jnp.int32, (KP, CAND), 0)
    expand = (je == ie).astype(jnp.float32)
    t_f = (lax.broadcasted_iota(jnp.int32, (_RT, CAND), 1) %
           BLK).astype(jnp.float32)
    gc = lax.dot_general(blk_f, expand, (((1,), (0,)), ((), ())),
                         preferred_element_type=jnp.float32) * float(BLK) + t_f
    big = jnp.float32(1e9)
    for k in range(K):
        mx = jnp.max(c, axis=1, keepdims=True)
        win = jnp.min(jnp.where(c == mx, gc, big), axis=1, keepdims=True)
        vals_ref[:, k:k + 1] = mx
        idx_ref[:, k:k + 1] = win.astype(jnp.int32)
        c = jnp.where((c == mx) & (gc == win), -jnp.inf, c)


def _final_topk(cands, g):
    return pl.pallas_call(
        _final_body,
        grid=(N // _RT,),
        in_specs=[
            pl.BlockSpec((_RT, CAND), lambda i: (i, 0)),
            pl.BlockSpec((_RT, KP), lambda i: (i, 0)),
        ],
        out_specs=[
            pl.BlockSpec((_RT, KP), lambda i: (i, 0)),
            pl.BlockSpec((_RT, KP), lambda i: (i, 0)),
        ],
        out_shape=[
            jax.ShapeDtypeStruct((N, KP), jnp.float32),
            jax.ShapeDtypeStruct((N, KP), jnp.int32),
        ],
    )(cands, g)


def kernel(X, Y):
    s3, m3 = _compute_scores(X, Y)
    m = m3.transpose(1, 0, 2).reshape(N, NB)
    mpad = jnp.pad(m, ((0, 0), (0, _MPAD - NB)), constant_values=-jnp.inf)
    g = _select_blocks(mpad)
    idx3 = g[:, :K].reshape(NW, CH, CW)
    s2 = s3.reshape(NB * N, BLK)
    cands = _sc_gather(s2, idx3)
    vals, idxs = _final_topk(cands.reshape(N, CAND), g)
    values = vals[:, :K]
    v = idxs[:, :K].reshape(-1)
    u = jnp.repeat(jnp.arange(N, dtype=jnp.int32), K)
    return (values, u, v)


# speedup vs baseline: 1.0882x; 1.0882x over previous
import functools

import jax
import jax.numpy as jnp
from jax import lax
from jax.experimental import pallas as pl
from jax.experimental.pallas import tpu as pltpu
from jax.experimental.pallas import tpu_sc as plsc

N = 1024
D = 128
MY = 100000
K = 30
BLK = 128
NB = 784
MP = NB * BLK
CB = 1024
GSTEPS = MP // CB
KP = 32
CAND = K * BLK
EPS = 1e-8

NC = 2
NS = 16
NW = NC * NS
BPW = (N * K) // NW
CH = 8
CW = BPW // CH


def _scores_body(x_ref, y_ref, s_ref, m_ref, xn_ref):
    g = pl.program_id(0)

    @pl.when(g == 0)
    def _():
        x = x_ref[...]
        nrm = jnp.sqrt(jnp.sum(x * x, axis=1, keepdims=True))
        xn_ref[...] = x / jnp.maximum(nrm, EPS)

    y = y_ref[...]
    ynrm = jnp.sqrt(jnp.sum(y * y, axis=1, keepdims=True))
    yn = y / jnp.maximum(ynrm, EPS)
    s = lax.dot_general(xn_ref[...], yn, (((1,), (1,)), ((), ())),
                        preferred_element_type=jnp.float32)
    cols = g * CB + lax.broadcasted_iota(jnp.int32, (N, CB), 1)
    s = jnp.where(cols < MY, s, -jnp.inf)
    maxes = []
    for b in range(CB // BLK):
        sb = s[:, b * BLK:(b + 1) * BLK]
        s_ref[b] = sb
        maxes.append(jnp.max(sb, axis=1, keepdims=True))
    m_ref[0] = jnp.concatenate(maxes, axis=1)


def _compute_scores(x, y):
    return pl.pallas_call(
        _scores_body,
        grid=(GSTEPS,),
        in_specs=[
            pl.BlockSpec((N, D), lambda g: (0, 0)),
            pl.BlockSpec((CB, D), lambda g: (g, 0)),
        ],
        out_specs=[
            pl.BlockSpec((CB // BLK, N, BLK), lambda g: (g, 0, 0)),
            pl.BlockSpec((1, N, CB // BLK), lambda g: (g, 0, 0)),
        ],
        out_shape=[
            jax.ShapeDtypeStruct((NB, N, BLK), jnp.float32),
            jax.ShapeDtypeStruct((GSTEPS, N, CB // BLK), jnp.float32),
        ],
        scratch_shapes=[pltpu.VMEM((N, D), jnp.float32)],
    )(x, y)


_MPAD = 896


def _blocksel_body(m_ref, g_ref):
    m = m_ref[...]
    iota = lax.broadcasted_iota(jnp.int32, (N, _MPAD), 1)
    rowc = lax.broadcasted_iota(jnp.int32, (N, 1), 0)
    g_ref[...] = lax.broadcasted_iota(jnp.int32, (N, KP), 0)
    big = jnp.int32(1 << 30)
    for k in range(K):
        mx = jnp.max(m, axis=1, keepdims=True)
        idx = jnp.min(jnp.where(m == mx, iota, big), axis=1, keepdims=True)
        g_ref[:, k:k + 1] = idx * N + rowc
        m = jnp.where(iota == idx, -jnp.inf, m)


def _select_blocks(mpad):
    return pl.pallas_call(
        _blocksel_body,
        out_shape=jax.ShapeDtypeStruct((N, KP), jnp.int32),
    )(mpad)


def _gather_body(s2_ref, idx_ref, out_ref, idx_v, rows_v, sem):
    wid = lax.axis_index("s") * NC + lax.axis_index("c")
    pltpu.sync_copy(idx_ref.at[wid], idx_v)
    cps = [pltpu.async_copy(s2_ref.at[idx_v.at[j]], rows_v.at[j], sem)
           for j in range(CH)]
    for cp in cps:
        cp.wait()
    pltpu.sync_copy(rows_v, out_ref.at[wid])


def _sc_gather(s2, idx3):
    mesh = plsc.VectorSubcoreMesh(core_axis_name="c", subcore_axis_name="s")
    fn = functools.partial(
        pl.kernel,
        mesh=mesh,
        out_type=jax.ShapeDtypeStruct((NW, CH, CW, BLK), jnp.float32),
        scratch_types=[
            pltpu.VMEM((CH, CW), jnp.int32),
            pltpu.VMEM((CH, CW, BLK), jnp.float32),
            pltpu.SemaphoreType.DMA,
        ],
    )(_gather_body)
    return fn(s2, idx3)


_RT = 128


def _final_body(c_ref, g_ref, vals_ref, idx_ref):
    c = c_ref[...]
    gsel = g_ref[...]
    blk_f = (gsel // N).astype(jnp.float32)
    je = lax.broadcasted_iota(jnp.int32, (KP, CAND), 1) // BLK
    ie = lax.broadcasted_iota(jnp.int32, (KP, CAND), 0)
    expand = (je == ie).astype(jnp.float32)
    t_f = (lax.broadcasted_iota(jnp.int32, (_RT, CAND), 1) %
           BLK).astype(jnp.float32)
    gc = lax.dot_general(blk_f, expand, (((1,), (0,)), ((), ())),
                         preferred_element_type=jnp.float32) * float(BLK) + t_f
    big = jnp.float32(1e9)
    for k in range(K):
        mx = jnp.max(c, axis=1, keepdims=True)
        win = jnp.min(jnp.where(c == mx, gc, big), axis=1, keepdims=True)
        vals_ref[:, k:k + 1] = mx
        idx_ref[:, k:k + 1] = win.astype(jnp.int32)
        c = jnp.where((c == mx) & (gc == win), -jnp.inf, c)


def _final_topk(cands, g):
    return pl.pallas_call(
        _final_body,
        grid=(N // _RT,),
        in_specs=[
            pl.BlockSpec((_RT, CAND), lambda i: (i, 0)),
            pl.BlockSpec((_RT, KP), lambda i: (i, 0)),
        ],
        out_specs=[
            pl.BlockSpec((_RT, KP), lambda i: (i, 0)),
            pl.BlockSpec((_RT, KP), lambda i: (i, 0)),
        ],
        out_shape=[
            jax.ShapeDtypeStruct((N, KP), jnp.float32),
            jax.ShapeDtypeStruct((N, KP), jnp.int32),
        ],
    )(cands, g)


def kernel(X, Y):
    s3, m3 = _compute_scores(X, Y)
    m = m3.transpose(1, 0, 2).reshape(N, NB)
    mpad = jnp.pad(m, ((0, 0), (0, _MPAD - NB)), constant_values=-jnp.inf)
    g = _select_blocks(mpad)
    idx3 = g[:, :K].reshape(NW, CH, CW)
    s2 = s3.reshape(NB * N, BLK)
    cands = _sc_gather(s2, idx3)
    vals, idxs = _final_topk(cands.reshape(N, CAND), g)
    values = vals[:, :K]
    v = idxs[:, :K].reshape(-1)
    u = jnp.repeat(jnp.arange(N, dtype=jnp.int32), K)
    return (values, u, v)


# speedup vs baseline: 1.2336x; 1.1336x over previous
import functools

import jax
import jax.numpy as jnp
from jax import lax
from jax.experimental import pallas as pl
from jax.experimental.pallas import tpu as pltpu
from jax.experimental.pallas import tpu_sc as plsc

N = 1024
D = 128
MY = 100000
K = 30
BLK = 128
NB = 784
MP = NB * BLK
CB = 2048
GSTEPS = MP // CB
KP = 32
CAND = K * BLK
EPS = 1e-8

NC = 2
NS = 16
NW = NC * NS
BPW = (N * K) // NW
CH = 8
CW = BPW // CH


def _scores_body(x_ref, y_ref, s_ref, m_ref, xn_ref):
    g = pl.program_id(0)

    @pl.when(g == 0)
    def _():
        x = x_ref[...]
        nrm = jnp.sqrt(jnp.sum(x * x, axis=1, keepdims=True))
        xn_ref[...] = x / jnp.maximum(nrm, EPS)

    y = y_ref[...]
    ynrm = jnp.sqrt(jnp.sum(y * y, axis=1, keepdims=True))
    yn = y / jnp.maximum(ynrm, EPS)
    s = lax.dot_general(xn_ref[...], yn, (((1,), (1,)), ((), ())),
                        preferred_element_type=jnp.float32)
    cols = g * CB + lax.broadcasted_iota(jnp.int32, (N, CB), 1)
    s = jnp.where(cols < MY, s, -jnp.inf)
    maxes = []
    for b in range(CB // BLK):
        sb = s[:, b * BLK:(b + 1) * BLK]
        s_ref[b] = sb
        maxes.append(jnp.max(sb, axis=1, keepdims=True))
    m_ref[0] = jnp.concatenate(maxes, axis=1)


def _compute_scores(x, y):
    return pl.pallas_call(
        _scores_body,
        grid=(GSTEPS,),
        in_specs=[
            pl.BlockSpec((N, D), lambda g: (0, 0)),
            pl.BlockSpec((CB, D), lambda g: (g, 0)),
        ],
        out_specs=[
            pl.BlockSpec((CB // BLK, N, BLK), lambda g: (g, 0, 0)),
            pl.BlockSpec((1, N, CB // BLK), lambda g: (g, 0, 0)),
        ],
        out_shape=[
            jax.ShapeDtypeStruct((NB, N, BLK), jnp.float32),
            jax.ShapeDtypeStruct((GSTEPS, N, CB // BLK), jnp.float32),
        ],
        scratch_shapes=[pltpu.VMEM((N, D), jnp.float32)],
    )(x, y)


_MPAD = 896


def _blocksel_body(m_ref, g_ref):
    m = m_ref[...]
    iota = lax.broadcasted_iota(jnp.int32, (N, _MPAD), 1)
    rowc = lax.broadcasted_iota(jnp.int32, (N, 1), 0)
    g_ref[...] = lax.broadcasted_iota(jnp.int32, (N, KP), 0)
    big = jnp.int32(1 << 30)
    for k in range(K):
        mx = jnp.max(m, axis=1, keepdims=True)
        idx = jnp.min(jnp.where(m == mx, iota, big), axis=1, keepdims=True)
        g_ref[:, k:k + 1] = idx * N + rowc
        m = jnp.where(iota == idx, -jnp.inf, m)


def _select_blocks(mpad):
    return pl.pallas_call(
        _blocksel_body,
        out_shape=jax.ShapeDtypeStruct((N, KP), jnp.int32),
    )(mpad)


def _gather_body(s2_ref, idx_ref, out_ref, idx_v, rows_v, sem):
    wid = lax.axis_index("s") * NC + lax.axis_index("c")
    pltpu.sync_copy(idx_ref.at[wid], idx_v)
    cps = [pltpu.async_copy(s2_ref.at[idx_v.at[j]], rows_v.at[j], sem)
           for j in range(CH)]
    for cp in cps:
        cp.wait()
    pltpu.sync_copy(rows_v, out_ref.at[wid])


def _sc_gather(s2, idx3):
    mesh = plsc.VectorSubcoreMesh(core_axis_name="c", subcore_axis_name="s")
    fn = functools.partial(
        pl.kernel,
        mesh=mesh,
        out_type=jax.ShapeDtypeStruct((NW, CH, CW, BLK), jnp.float32),
        scratch_types=[
            pltpu.VMEM((CH, CW), jnp.int32),
            pltpu.VMEM((CH, CW, BLK), jnp.float32),
            pltpu.SemaphoreType.DMA,
        ],
    )(_gather_body)
    return fn(s2, idx3)


_RT = 128


def _final_body(c_ref, g_ref, vals_ref, idx_ref):
    c = c_ref[...]
    gsel = g_ref[...]
    blk_f = (gsel // N).astype(jnp.float32)
    je = lax.broadcasted_iota(jnp.int32, (KP, CAND), 1) // BLK
    ie = lax.broadcasted_iota(jnp.int32, (KP, CAND), 0)
    expand = (je == ie).astype(jnp.float32)
    t_f = (lax.broadcasted_iota(jnp.int32, (_RT, CAND), 1) %
           BLK).astype(jnp.float32)
    gc = lax.dot_general(blk_f, expand, (((1,), (0,)), ((), ())),
                         preferred_element_type=jnp.float32) * float(BLK) + t_f
    big = jnp.float32(1e9)
    for k in range(K):
        mx = jnp.max(c, axis=1, keepdims=True)
        win = jnp.min(jnp.where(c == mx, gc, big), axis=1, keepdims=True)
        vals_ref[:, k:k + 1] = mx
        idx_ref[:, k:k + 1] = win.astype(jnp.int32)
        c = jnp.where((c == mx) & (gc == win), -jnp.inf, c)


def _final_topk(cands, g):
    return pl.pallas_call(
        _final_body,
        grid=(N // _RT,),
        in_specs=[
            pl.BlockSpec((_RT, CAND), lambda i: (i, 0)),
            pl.BlockSpec((_RT, KP), lambda i: (i, 0)),
        ],
        out_specs=[
            pl.BlockSpec((_RT, KP), lambda i: (i, 0)),
            pl.BlockSpec((_RT, KP), lambda i: (i, 0)),
        ],
        out_shape=[
            jax.ShapeDtypeStruct((N, KP), jnp.float32),
            jax.ShapeDtypeStruct((N, KP), jnp.int32),
        ],
    )(cands, g)


def kernel(X, Y):
    s3, m3 = _compute_scores(X, Y)
    m = m3.transpose(1, 0, 2).reshape(N, NB)
    mpad = jnp.pad(m, ((0, 0), (0, _MPAD - NB)), constant_values=-jnp.inf)
    g = _select_blocks(mpad)
    idx3 = g[:, :K].reshape(NW, CH, CW)
    s2 = s3.reshape(NB * N, BLK)
    cands = _sc_gather(s2, idx3)
    vals, idxs = _final_topk(cands.reshape(N, CAND), g)
    values = vals[:, :K]
    v = idxs[:, :K].reshape(-1)
    u = jnp.repeat(jnp.arange(N, dtype=jnp.int32), K)
    return (values, u, v)


# speedup vs baseline: 1.2644x; 1.0250x over previous
import functools

import jax
import jax.numpy as jnp
from jax import lax
from jax.experimental import pallas as pl
from jax.experimental.pallas import tpu as pltpu
from jax.experimental.pallas import tpu_sc as plsc

N = 1024
D = 128
MY = 100000
K = 30
BLK = 128
NB = 784
MP = NB * BLK
CB = 3584
GSTEPS = MP // CB
KP = 32
CAND = K * BLK
EPS = 1e-8

NC = 2
NS = 16
NW = NC * NS
BPW = (N * K) // NW
CH = 8
CW = BPW // CH


def _scores_body(x_ref, y_ref, s_ref, m_ref, xn_ref):
    g = pl.program_id(0)

    @pl.when(g == 0)
    def _():
        x = x_ref[...]
        nrm = jnp.sqrt(jnp.sum(x * x, axis=1, keepdims=True))
        xn_ref[...] = x / jnp.maximum(nrm, EPS)

    y = y_ref[...]
    ynrm = jnp.sqrt(jnp.sum(y * y, axis=1, keepdims=True))
    yn = y / jnp.maximum(ynrm, EPS)
    s = lax.dot_general(xn_ref[...], yn, (((1,), (1,)), ((), ())),
                        preferred_element_type=jnp.float32)
    cols = g * CB + lax.broadcasted_iota(jnp.int32, (N, CB), 1)
    s = jnp.where(cols < MY, s, -jnp.inf)
    maxes = []
    for b in range(CB // BLK):
        sb = s[:, b * BLK:(b + 1) * BLK]
        s_ref[b] = sb
        maxes.append(jnp.max(sb, axis=1, keepdims=True))
    m_ref[0] = jnp.concatenate(maxes, axis=1)


def _compute_scores(x, y):
    return pl.pallas_call(
        _scores_body,
        grid=(GSTEPS,),
        in_specs=[
            pl.BlockSpec((N, D), lambda g: (0, 0)),
            pl.BlockSpec((CB, D), lambda g: (g, 0)),
        ],
        out_specs=[
            pl.BlockSpec((CB // BLK, N, BLK), lambda g: (g, 0, 0)),
            pl.BlockSpec((1, N, CB // BLK), lambda g: (g, 0, 0)),
        ],
        out_shape=[
            jax.ShapeDtypeStruct((NB, N, BLK), jnp.float32),
            jax.ShapeDtypeStruct((GSTEPS, N, CB // BLK), jnp.float32),
        ],
        scratch_shapes=[pltpu.VMEM((N, D), jnp.float32)],
    )(x, y)


_MPAD = 896


def _blocksel_body(m_ref, g_ref):
    m = m_ref[...]
    iota = lax.broadcasted_iota(jnp.int32, (N, _MPAD), 1)
    rowc = lax.broadcasted_iota(jnp.int32, (N, 1), 0)
    g_ref[...] = lax.broadcasted_iota(jnp.int32, (N, KP), 0)
    big = jnp.int32(1 << 30)
    for k in range(K):
        mx = jnp.max(m, axis=1, keepdims=True)
        idx = jnp.min(jnp.where(m == mx, iota, big), axis=1, keepdims=True)
        g_ref[:, k:k + 1] = idx * N + rowc
        m = jnp.where(iota == idx, -jnp.inf, m)


def _select_blocks(mpad):
    return pl.pallas_call(
        _blocksel_body,
        out_shape=jax.ShapeDtypeStruct((N, KP), jnp.int32),
    )(mpad)


def _gather_body(s2_ref, idx_ref, out_ref, idx_v, rows_v, sem):
    wid = lax.axis_index("s") * NC + lax.axis_index("c")
    pltpu.sync_copy(idx_ref.at[wid], idx_v)
    cps = [pltpu.async_copy(s2_ref.at[idx_v.at[j]], rows_v.at[j], sem)
           for j in range(CH)]
    for cp in cps:
        cp.wait()
    pltpu.sync_copy(rows_v, out_ref.at[wid])


def _sc_gather(s2, idx3):
    mesh = plsc.VectorSubcoreMesh(core_axis_name="c", subcore_axis_name="s")
    fn = functools.partial(
        pl.kernel,
        mesh=mesh,
        out_type=jax.ShapeDtypeStruct((NW, CH, CW, BLK), jnp.float32),
        scratch_types=[
            pltpu.VMEM((CH, CW), jnp.int32),
            pltpu.VMEM((CH, CW, BLK), jnp.float32),
            pltpu.SemaphoreType.DMA,
        ],
    )(_gather_body)
    return fn(s2, idx3)


_RT = 128


def _final_body(c_ref, g_ref, vals_ref, idx_ref):
    c = c_ref[...]
    gsel = g_ref[...]
    blk_f = (gsel // N).astype(jnp.float32)
    je = lax.broadcasted_iota(jnp.int32, (KP, CAND), 1) // BLK
    ie = lax.broadcasted_iota(jnp.int32, (KP, CAND), 0)
    expand = (je == ie).astype(jnp.float32)
    t_f = (lax.broadcasted_iota(jnp.int32, (_RT, CAND), 1) %
           BLK).astype(jnp.float32)
    gc = lax.dot_general(blk_f, expand, (((1,), (0,)), ((), ())),
                         preferred_element_type=jnp.float32) * float(BLK) + t_f
    big = jnp.float32(1e9)
    for k in range(K):
        mx = jnp.max(c, axis=1, keepdims=True)
        win = jnp.min(jnp.where(c == mx, gc, big), axis=1, keepdims=True)
        vals_ref[:, k:k + 1] = mx
        idx_ref[:, k:k + 1] = win.astype(jnp.int32)
        c = jnp.where((c == mx) & (gc == win), -jnp.inf, c)


def _final_topk(cands, g):
    return pl.pallas_call(
        _final_body,
        grid=(N // _RT,),
        in_specs=[
            pl.BlockSpec((_RT, CAND), lambda i: (i, 0)),
            pl.BlockSpec((_RT, KP), lambda i: (i, 0)),
        ],
        out_specs=[
            pl.BlockSpec((_RT, KP), lambda i: (i, 0)),
            pl.BlockSpec((_RT, KP), lambda i: (i, 0)),
        ],
        out_shape=[
            jax.ShapeDtypeStruct((N, KP), jnp.float32),
            jax.ShapeDtypeStruct((N, KP), jnp.int32),
        ],
    )(cands, g)


def kernel(X, Y):
    s3, m3 = _compute_scores(X, Y)
    m = m3.transpose(1, 0, 2).reshape(N, NB)
    mpad = jnp.pad(m, ((0, 0), (0, _MPAD - NB)), constant_values=-jnp.inf)
    g = _select_blocks(mpad)
    idx3 = g[:, :K].reshape(NW, CH, CW)
    s2 = s3.reshape(NB * N, BLK)
    cands = _sc_gather(s2, idx3)
    vals, idxs = _final_topk(cands.reshape(N, CAND), g)
    values = vals[:, :K]
    v = idxs[:, :K].reshape(-1)
    u = jnp.repeat(jnp.arange(N, dtype=jnp.int32), K)
    return (values, u, v)


# speedup vs baseline: 1.3239x; 1.0471x over previous
import functools

import jax
import jax.numpy as jnp
from jax import lax
from jax.experimental import pallas as pl
from jax.experimental.pallas import tpu as pltpu
from jax.experimental.pallas import tpu_sc as plsc

N = 1024
D = 128
MY = 100000
K = 30
BLK = 128
NB = 784
MP = NB * BLK
CB = 3584
GSTEPS = MP // CB
KP = 32
CAND = K * BLK
EPS = 1e-8

NC = 2
NS = 16
NW = NC * NS
BPW = (N * K) // NW
CH = 8
CW = BPW // CH


def _scores_body(x_ref, y_ref, s_ref, m_ref, xn_ref):
    g = pl.program_id(0)

    @pl.when(g == 0)
    def _():
        x = x_ref[...]
        nrm = jnp.sqrt(jnp.sum(x * x, axis=1, keepdims=True))
        xn_ref[...] = x / jnp.maximum(nrm, EPS)

    y = y_ref[...]
    ynrm = jnp.sqrt(jnp.sum(y * y, axis=1, keepdims=True))
    yn = y / jnp.maximum(ynrm, EPS)
    s = lax.dot_general(xn_ref[...], yn, (((1,), (1,)), ((), ())),
                        preferred_element_type=jnp.float32)
    cols = g * CB + lax.broadcasted_iota(jnp.int32, (N, CB), 1)
    s = jnp.where(cols < MY, s, -jnp.inf)
    maxes = []
    for b in range(CB // BLK):
        sb = s[:, b * BLK:(b + 1) * BLK]
        s_ref[b] = sb
        maxes.append(jnp.max(sb, axis=1, keepdims=True))
    m_ref[0] = jnp.concatenate(maxes, axis=1)


def _compute_scores(x, y):
    return pl.pallas_call(
        _scores_body,
        grid=(GSTEPS,),
        in_specs=[
            pl.BlockSpec((N, D), lambda g: (0, 0)),
            pl.BlockSpec((CB, D), lambda g: (g, 0)),
        ],
        out_specs=[
            pl.BlockSpec((CB // BLK, N, BLK), lambda g: (g, 0, 0)),
            pl.BlockSpec((1, N, CB // BLK), lambda g: (g, 0, 0)),
        ],
        out_shape=[
            jax.ShapeDtypeStruct((NB, N, BLK), jnp.float32),
            jax.ShapeDtypeStruct((GSTEPS, N, CB // BLK), jnp.float32),
        ],
        scratch_shapes=[pltpu.VMEM((N, D), jnp.float32)],
    )(x, y)


_MPAD = 896


def _blocksel_body(m_ref, g_ref):
    m = m_ref[...]
    iota = lax.broadcasted_iota(jnp.int32, (N, _MPAD), 1)
    rowc = lax.broadcasted_iota(jnp.int32, (N, 1), 0)
    g_ref[...] = lax.broadcasted_iota(jnp.int32, (N, KP), 0)
    big = jnp.int32(1 << 30)
    for k in range(K):
        mx = jnp.max(m, axis=1, keepdims=True)
        idx = jnp.min(jnp.where(m == mx, iota, big), axis=1, keepdims=True)
        g_ref[:, k:k + 1] = idx * N + rowc
        m = jnp.where(iota == idx, -jnp.inf, m)


def _select_blocks(mpad):
    return pl.pallas_call(
        _blocksel_body,
        out_shape=jax.ShapeDtypeStruct((N, KP), jnp.int32),
    )(mpad)


def _gather_body(s2_ref, idx_ref, out_ref, idx_v, rows_v, sem):
    wid = lax.axis_index("s") * NC + lax.axis_index("c")
    pltpu.sync_copy(idx_ref.at[wid], idx_v)
    cps = [pltpu.async_copy(s2_ref.at[idx_v.at[j]], rows_v.at[j], sem)
           for j in range(CH)]
    for cp in cps:
        cp.wait()
    pltpu.sync_copy(rows_v, out_ref.at[wid])


def _sc_gather(s2, idx3):
    mesh = plsc.VectorSubcoreMesh(core_axis_name="c", subcore_axis_name="s")
    fn = functools.partial(
        pl.kernel,
        mesh=mesh,
        out_type=jax.ShapeDtypeStruct((NW, CH, CW, BLK), jnp.float32),
        scratch_types=[
            pltpu.VMEM((CH, CW), jnp.int32),
            pltpu.VMEM((CH, CW, BLK), jnp.float32),
            pltpu.SemaphoreType.DMA,
        ],
    )(_gather_body)
    return fn(s2, idx3)


_RT = 128


def _final_body(c_ref, g_ref, vals_ref, idx_ref):
    c = c_ref[...]
    gsel = g_ref[...]
    blk_f = (gsel // N).astype(jnp.float32)
    je = lax.broadcasted_iota(jnp.int32, (KP, CAND), 1) // BLK
    ie = lax.broadcasted_iota(jnp.int32, (KP, CAND), 0)
    expand = (je == ie).astype(jnp.float32)
    t_f = (lax.broadcasted_iota(jnp.int32, (_RT, CAND), 1) %
           BLK).astype(jnp.float32)
    gc = lax.dot_general(blk_f, expand, (((1,), (0,)), ((), ())),
                         preferred_element_type=jnp.float32) * float(BLK) + t_f
    big = jnp.float32(1e9)
    for k in range(K):
        mx = jnp.max(c, axis=1, keepdims=True)
        win = jnp.min(jnp.where(c == mx, gc, big), axis=1, keepdims=True)
        vals_ref[:, k:k + 1] = mx
        idx_ref[:, k:k + 1] = win.astype(jnp.int32)
        c = jnp.where(gc == win, -jnp.inf, c)


def _final_topk(cands, g):
    return pl.pallas_call(
        _final_body,
        grid=(N // _RT,),
        in_specs=[
            pl.BlockSpec((_RT, CAND), lambda i: (i, 0)),
            pl.BlockSpec((_RT, KP), lambda i: (i, 0)),
        ],
        out_specs=[
            pl.BlockSpec((_RT, KP), lambda i: (i, 0)),
            pl.BlockSpec((_RT, KP), lambda i: (i, 0)),
        ],
        out_shape=[
            jax.ShapeDtypeStruct((N, KP), jnp.float32),
            jax.ShapeDtypeStruct((N, KP), jnp.int32),
        ],
    )(cands, g)


def kernel(X, Y):
    s3, m3 = _compute_scores(X, Y)
    m = m3.transpose(1, 0, 2).reshape(N, NB)
    mpad = jnp.pad(m, ((0, 0), (0, _MPAD - NB)), constant_values=-jnp.inf)
    g = _select_blocks(mpad)
    idx3 = g[:, :K].reshape(NW, CH, CW)
    s2 = s3.reshape(NB * N, BLK)
    cands = _sc_gather(s2, idx3)
    vals, idxs = _final_topk(cands.reshape(N, CAND), g)
    values = vals[:, :K]
    v = idxs[:, :K].reshape(-1)
    u = jnp.repeat(jnp.arange(N, dtype=jnp.int32), K)
    return (values, u, v)
